# baseline (speedup 1.0000x reference)
import functools

import jax
import jax.numpy as jnp
from jax import lax
from jax.experimental import pallas as pl
from jax.experimental.pallas import tpu as pltpu
from jax.experimental.pallas import tpu_sc as plsc

N = 100000
D = 128
E = 320000
NUM_REL = 2

NC = 2
NS = 16
NW = NC * NS
C = 80
W = D // 2
KB = W // 16
ROWS = E // C
WROWS = ROWS // NW
GROUPS = C // 16
HIMASK = -65536
REORDER = (0, 8, 4, 12, 2, 10, 6, 14, 1, 9, 5, 13, 3, 11, 7, 15)


def _sc_body(emb, rel, sidx0, didx0, sidx1, didx1, sidx2, didx2,
             sidx3, didx3, out, sidx_v, didx_v, srows0, drows0,
             srows1, drows1, rel_v, scores_v, sem_g):
    wid = lax.axis_index("s") * NC + lax.axis_index("c")

    iota = lax.iota(jnp.int32, 16)
    _gdn = lax.GatherDimensionNumbers(offset_dims=(),
                                      collapsed_slice_dims=(0,),
                                      start_index_map=(0,))

    def _perm(x, perm):
        return lax.gather(x, perm[:, None], _gdn, slice_sizes=(1,),
                          mode=lax.GatherScatterMode.PROMISE_IN_BOUNDS)

    xperms = {sh: iota ^ sh for sh in (8, 4, 2, 1)}
    masks = {sh: (iota & sh) == 0 for sh in (8, 4, 2, 1)}
    reorder = (((iota & 1) << 3) | ((iota & 2) << 1)
               | ((iota & 4) >> 1) | ((iota & 8) >> 3))

    def combine(u, v, sh):
        pu = _perm(u, xperms[sh])
        pv = _perm(v, xperms[sh])
        m = masks[sh]
        return jnp.where(m, u, pv) + jnp.where(m, pu, v)

    src_refs = (sidx0, sidx1, sidx2, sidx3)
    dst_refs = (didx0, didx1, didx2, didx3)

    pltpu.sync_copy(rel, rel_v)

    def compute(ci, sbuf, dbuf, rve, rvo):
        def gbody(g, carry):
            halves = []
            for h in range(2):
                vs = []
                for j in range(8):
                    e = g * 16 + h * 8 + j
                    acc_e = None
                    acc_o = None
                    for k in range(KB):
                        sw = sbuf[e, pl.ds(k * 16, 16)]
                        dw = dbuf[e, pl.ds(k * 16, 16)]
                        se = lax.bitcast_convert_type(sw << 16,
                                                      jnp.float32)
                        so = lax.bitcast_convert_type(sw & HIMASK,
                                                      jnp.float32)
                        de = lax.bitcast_convert_type(dw << 16,
                                                      jnp.float32)
                        do = lax.bitcast_convert_type(dw & HIMASK,
                                                      jnp.float32)
                        if acc_e is None:
                            acc_e = se * de * rve[k]
                            acc_o = so * do * rvo[k]
                        else:
                            acc_e = acc_e + se * de * rve[k]
                            acc_o = acc_o + so * do * rvo[k]
                    vs.append(acc_e + acc_o)
                for sh in (8, 4, 2):
                    vs = [combine(vs[2 * i], vs[2 * i + 1], sh)
                          for i in range(len(vs) // 2)]
                halves.append(vs[0])
            final = combine(halves[0], halves[1], 1)
            scores_v[ci, pl.ds(g * 16, 16)] = _perm(final, reorder)
            return carry

        lax.fori_loop(0, GROUPS, gbody, 0)

    def set_body(t, carry):
        for tt in range(4):
            @pl.when(t == tt)
            def _():
                pltpu.sync_copy(src_refs[tt].at[wid], sidx_v)
                pltpu.sync_copy(dst_refs[tt].at[wid], didx_v)

        rel_row = t // 2
        rve = [rel_v[rel_row, 0, k, pl.ds(0, 16)] for k in range(KB)]
        rvo = [rel_v[rel_row, 1, k, pl.ds(0, 16)] for k in range(KB)]

        def fire(ci, sbuf, dbuf, b):
            pltpu.async_copy(emb.at[sidx_v.at[ci]], sbuf, sem_g.at[b])
            pltpu.async_copy(emb.at[didx_v.at[ci]], dbuf, sem_g.at[b])

        def drain(ci, sbuf, dbuf, b):
            pltpu.make_async_copy(emb.at[sidx_v.at[ci]], sbuf,
                                  sem_g.at[b]).wait()
            pltpu.make_async_copy(emb.at[didx_v.at[ci]], dbuf,
                                  sem_g.at[b]).wait()

        fire(0, srows0, drows0, 0)

        def pair_body(i, carry2):
            c0 = 2 * i
            fire(c0 + 1, srows1, drows1, 1)
            drain(c0, srows0, drows0, 0)
            compute(c0, srows0, drows0, rve, rvo)
            fire(c0 + 2, srows0, drows0, 0)
            drain(c0 + 1, srows1, drows1, 1)
            compute(c0 + 1, srows1, drows1, rve, rvo)
            return carry2

        lax.fori_loop(0, (WROWS - 1) // 2, pair_body, 0)
        drain(WROWS - 1, srows0, drows0, 0)
        compute(WROWS - 1, srows0, drows0, rve, rvo)

        pltpu.sync_copy(scores_v, out.at[t, wid])
        return carry

    lax.fori_loop(0, 4, set_body, 0)


@functools.partial(
    pl.kernel,
    out_type=jax.ShapeDtypeStruct((4, NW, WROWS, C), jnp.float32),
    mesh=plsc.VectorSubcoreMesh(core_axis_name="c", subcore_axis_name="s",
                                num_cores=NC, num_subcores=NS),
    compiler_params=pltpu.CompilerParams(use_tc_tiling_on_sc=False),
    scratch_types=[
        pltpu.VMEM((WROWS, C), jnp.int32),
        pltpu.VMEM((WROWS, C), jnp.int32),
        pltpu.VMEM((C, W), jnp.int32),
        pltpu.VMEM((C, W), jnp.int32),
        pltpu.VMEM((C, W), jnp.int32),
        pltpu.VMEM((C, W), jnp.int32),
        pltpu.VMEM((NUM_REL, 2, KB, 16), jnp.float32),
        pltpu.VMEM((WROWS, C), jnp.float32),
        pltpu.SemaphoreType.DMA((2,)),
    ],
)
def _sc_kernel(*args):
    _sc_body(*args)


N2 = N // 2


def _pack_rows(x):
    eb = x.astype(jnp.bfloat16)
    lo = lax.bitcast_convert_type(eb[:, :W], jnp.uint16).astype(jnp.uint32)
    hi = lax.bitcast_convert_type(eb[:, W:], jnp.uint16).astype(jnp.uint32)
    return lax.bitcast_convert_type(lo | (hi << jnp.uint32(16)), jnp.int32)


def _pack(x):
    w2 = jnp.concatenate([_pack_rows(x[:N2]), _pack_rows(x[N2:])], axis=1)
    return w2.reshape(N, W)


def kernel(embeddings, relation_weights, pos_src_interacts,
           pos_dst_interacts, neg_src_interacts, neg_dst_interacts,
           pos_src_regulates, pos_dst_regulates, neg_src_regulates,
           neg_dst_regulates):
    emb_pk = _pack(embeddings)
    rel_de = relation_weights.reshape(NUM_REL, 2, KB, 16)
    def _remap(a):
        a = jnp.asarray(a, jnp.int32)
        return jnp.where(a < N2, 2 * a, 2 * (a - N2) + 1).reshape(
            NW, WROWS, C)

    idx = [
        _remap(a)
        for a in (pos_src_interacts, pos_dst_interacts,
                  neg_src_interacts, neg_dst_interacts,
                  pos_src_regulates, pos_dst_regulates,
                  neg_src_regulates, neg_dst_regulates)
    ]
    out = _sc_kernel(emb_pk, rel_de, *idx)
    return out.reshape(4, E)


# speedup vs baseline: 1.0096x; 1.0096x over previous
import functools

import jax
import jax.numpy as jnp
from jax import lax
from jax.experimental import pallas as pl
from jax.experimental.pallas import tpu as pltpu
from jax.experimental.pallas import tpu_sc as plsc

N = 100000
D = 128
E = 320000
NUM_REL = 2

NC = 2
NS = 16
NW = NC * NS
C = 80
W = D // 2
KB = W // 16
ROWS = E // C
WROWS = ROWS // NW
GROUPS = C // 16
HIMASK = -65536
REORDER = (0, 8, 4, 12, 2, 10, 6, 14, 1, 9, 5, 13, 3, 11, 7, 15)


def _sc_body(emb, rel, sidx0, didx0, sidx1, didx1, sidx2, didx2,
             sidx3, didx3, out, sidx_v, didx_v, srows0, drows0,
             srows1, drows1, rel_v, scores_v, sem_g):
    wid = lax.axis_index("s") * NC + lax.axis_index("c")

    iota = lax.iota(jnp.int32, 16)
    _gdn = lax.GatherDimensionNumbers(offset_dims=(),
                                      collapsed_slice_dims=(0,),
                                      start_index_map=(0,))

    def _perm(x, perm):
        return lax.gather(x, perm[:, None], _gdn, slice_sizes=(1,),
                          mode=lax.GatherScatterMode.PROMISE_IN_BOUNDS)

    xperms = {sh: iota ^ sh for sh in (8, 4, 2, 1)}
    masks = {sh: (iota & sh) == 0 for sh in (8, 4, 2, 1)}
    reorder = (((iota & 1) << 3) | ((iota & 2) << 1)
               | ((iota & 4) >> 1) | ((iota & 8) >> 3))

    def combine(u, v, sh):
        pu = _perm(u, xperms[sh])
        pv = _perm(v, xperms[sh])
        m = masks[sh]
        return jnp.where(m, u, pv) + jnp.where(m, pu, v)

    src_refs = (sidx0, sidx1, sidx2, sidx3)
    dst_refs = (didx0, didx1, didx2, didx3)

    pltpu.sync_copy(rel, rel_v)

    def compute(ci, sbuf, dbuf, rve, rvo):
        def gbody(g, carry):
            halves = []
            for h in range(2):
                vs = []
                for j in range(8):
                    e = g * 16 + h * 8 + j
                    acc_e = None
                    acc_o = None
                    for k in range(KB):
                        sw = sbuf[e, pl.ds(k * 16, 16)]
                        dw = dbuf[e, pl.ds(k * 16, 16)]
                        se = lax.bitcast_convert_type(sw << 16,
                                                      jnp.float32)
                        so = lax.bitcast_convert_type(sw & HIMASK,
                                                      jnp.float32)
                        de = lax.bitcast_convert_type(dw << 16,
                                                      jnp.float32)
                        do = lax.bitcast_convert_type(dw & HIMASK,
                                                      jnp.float32)
                        if acc_e is None:
                            acc_e = se * de * rve[k]
                            acc_o = so * do * rvo[k]
                        else:
                            acc_e = acc_e + se * de * rve[k]
                            acc_o = acc_o + so * do * rvo[k]
                    vs.append(acc_e + acc_o)
                for sh in (8, 4, 2):
                    vs = [combine(vs[2 * i], vs[2 * i + 1], sh)
                          for i in range(len(vs) // 2)]
                halves.append(vs[0])
            final = combine(halves[0], halves[1], 1)
            scores_v[ci, pl.ds(g * 16, 16)] = _perm(final, reorder)
            return carry

        lax.fori_loop(0, GROUPS, gbody, 0)

    def set_body(t, carry):
        for tt in range(4):
            @pl.when(t == tt)
            def _():
                pltpu.sync_copy(src_refs[tt].at[wid], sidx_v)
                pltpu.sync_copy(dst_refs[tt].at[wid], didx_v)

        rel_row = t // 2
        rve = [rel_v[rel_row, 0, k, pl.ds(0, 16)] for k in range(KB)]
        rvo = [rel_v[rel_row, 1, k, pl.ds(0, 16)] for k in range(KB)]

        def fire(ci, sbuf, dbuf, b):
            pltpu.async_copy(emb.at[sidx_v.at[ci]], sbuf, sem_g.at[b])
            pltpu.async_copy(emb.at[didx_v.at[ci]], dbuf, sem_g.at[b])

        def drain(ci, sbuf, dbuf, b):
            pltpu.make_async_copy(emb.at[sidx_v.at[ci]], sbuf,
                                  sem_g.at[b]).wait()
            pltpu.make_async_copy(emb.at[didx_v.at[ci]], dbuf,
                                  sem_g.at[b]).wait()

        fire(0, srows0, drows0, 0)

        def pair_body(i, carry2):
            c0 = 2 * i
            fire(c0 + 1, srows1, drows1, 1)
            drain(c0, srows0, drows0, 0)
            compute(c0, srows0, drows0, rve, rvo)
            fire(c0 + 2, srows0, drows0, 0)
            drain(c0 + 1, srows1, drows1, 1)
            compute(c0 + 1, srows1, drows1, rve, rvo)
            return carry2

        lax.fori_loop(0, (WROWS - 1) // 2, pair_body, 0)
        drain(WROWS - 1, srows0, drows0, 0)
        compute(WROWS - 1, srows0, drows0, rve, rvo)

        pltpu.sync_copy(scores_v, out.at[t, wid])
        return carry

    lax.fori_loop(0, 4, set_body, 0)


@functools.partial(
    pl.kernel,
    out_type=jax.ShapeDtypeStruct((4, NW, WROWS, C), jnp.float32),
    mesh=plsc.VectorSubcoreMesh(core_axis_name="c", subcore_axis_name="s",
                                num_cores=NC, num_subcores=NS),
    compiler_params=pltpu.CompilerParams(use_tc_tiling_on_sc=False),
    scratch_types=[
        pltpu.VMEM((WROWS, C), jnp.int32),
        pltpu.VMEM((WROWS, C), jnp.int32),
        pltpu.VMEM((C, W), jnp.int32),
        pltpu.VMEM((C, W), jnp.int32),
        pltpu.VMEM((C, W), jnp.int32),
        pltpu.VMEM((C, W), jnp.int32),
        pltpu.VMEM((NUM_REL, 2, KB, 16), jnp.float32),
        pltpu.VMEM((WROWS, C), jnp.float32),
        pltpu.SemaphoreType.DMA((2,)),
    ],
)
def _sc_kernel(*args):
    _sc_body(*args)


def _pack(x):
    eb = x.astype(jnp.bfloat16)
    lo = lax.bitcast_convert_type(eb[:, :W], jnp.uint16).astype(jnp.uint32)
    hi = lax.bitcast_convert_type(eb[:, W:], jnp.uint16).astype(jnp.uint32)
    return lax.bitcast_convert_type(lo | (hi << jnp.uint32(16)), jnp.int32)


def kernel(embeddings, relation_weights, pos_src_interacts,
           pos_dst_interacts, neg_src_interacts, neg_dst_interacts,
           pos_src_regulates, pos_dst_regulates, neg_src_regulates,
           neg_dst_regulates):
    emb_pk = _pack(embeddings)
    rel_de = relation_weights.reshape(NUM_REL, 2, KB, 16)
    idx = [
        jnp.asarray(a, jnp.int32).reshape(NW, WROWS, C)
        for a in (pos_src_interacts, pos_dst_interacts,
                  neg_src_interacts, neg_dst_interacts,
                  pos_src_regulates, pos_dst_regulates,
                  neg_src_regulates, neg_dst_regulates)
    ]
    out = _sc_kernel(emb_pk, rel_de, *idx)
    return out.reshape(4, E)
